# Initial kernel scaffold; baseline (speedup 1.0000x reference)
import jax
import jax.numpy as jnp
from jax.experimental import pallas as pl


def kernel(edge_index, atom_types, edge_lengths, edge_cutoff, sigma, delta, epsilon):
    raise NotImplementedError("write your pallas kernel here")


# speedup vs baseline: 83.1190x; 83.1190x over previous
import functools

import jax
import jax.numpy as jnp
from jax import lax
from jax.experimental import pallas as pl
from jax.experimental.pallas import tpu as pltpu
from jax.experimental.pallas import tpu_sc as plsc

N_NODES = 100000
N_EDGES = 6400000
NUM_TYPES = 8

NC = 2
NS = 16
NW = NC * NS
EPW = N_EDGES // NW
CHUNK = 400
NCHUNK = EPW // CHUNK
ROWS = 784
ROWS_PER_TILE = ROWS // NS
VPC = CHUNK // 16


def _prep_table(raw_hbm, stage_v, tab_v, scale):
    pltpu.sync_copy(raw_hbm, stage_v)
    for g in range(4):
        k = lax.iota(jnp.int32, 16) + g * 16
        i = lax.shift_right_logical(k, 3)
        j = k & 7
        row = jnp.minimum(i, j)
        col = jnp.maximum(i, j)
        v = plsc.load_gather(stage_v, [lax.shift_left(row, 3) | col])
        v = jnp.maximum(v, 0.0) * scale
        tab_v[pl.ds(g * 16, 16)] = v


def _sc_body(dst_hbm, src_hbm, tp_hbm, len_hbm, cut_hbm, sig_hbm, del_hbm, eps_hbm,
             out_hbm,
             acc_v, tp_v, stab_v, dtab_v, etab_v, stage_v,
             bdst_v, bsrc_v, blen_v, bcut_v):
    cid = lax.axis_index("c")
    sid = lax.axis_index("s")
    wid = cid * NS + sid

    pltpu.sync_copy(tp_hbm, tp_v)
    _prep_table(sig_hbm, stage_v, stab_v, 1.0)
    _prep_table(del_hbm, stage_v, dtab_v, 1.0)
    _prep_table(eps_hbm, stage_v, etab_v, 2.0)

    zf = jnp.zeros((16,), jnp.float32)

    def zbody(r, _):
        for k in range(8):
            acc_v[r, pl.ds(k * 16, 16)] = zf
        return 0

    lax.fori_loop(0, ROWS, zbody, 0)

    base = wid * EPW

    def chunk_body(ci, _):
        off = base + ci * CHUNK
        pltpu.sync_copy(dst_hbm.at[pl.ds(off, CHUNK)], bdst_v)
        pltpu.sync_copy(src_hbm.at[pl.ds(off, CHUNK)], bsrc_v)
        pltpu.sync_copy(len_hbm.at[pl.ds(off, CHUNK)], blen_v)
        pltpu.sync_copy(cut_hbm.at[pl.ds(off, CHUNK)], bcut_v)

        def vec_body(v, _):
            sl = pl.ds(v * 16, 16)
            dst = bdst_v[sl]
            src = bsrc_v[sl]
            w1 = plsc.load_gather(tp_v, [lax.shift_right_logical(dst, 2)])
            t1 = lax.shift_right_logical(w1, lax.shift_left(dst & 3, 3)) & 7
            w2 = plsc.load_gather(tp_v, [lax.shift_right_logical(src, 2)])
            t2 = lax.shift_right_logical(w2, lax.shift_left(src & 3, 3)) & 7
            pidx = lax.shift_left(t1, 3) | t2
            s = plsc.load_gather(stab_v, [pidx])
            d = plsc.load_gather(dtab_v, [pidx])
            e2 = plsc.load_gather(etab_v, [pidx])
            ln = blen_v[sl]
            ct = bcut_v[sl]
            b = s / (ln - d)
            b2 = b * b
            p = b2 * b2 * b2
            val = e2 * ct * (p * p - p)
            row = lax.shift_right_logical(dst, 7)
            col = dst & 127
            plsc.addupdate_scatter(acc_v, [row, col], val)
            return 0

        lax.fori_loop(0, VPC, vec_body, 0)
        return 0

    lax.fori_loop(0, NCHUNK, chunk_body, 0)

    pltpu.sync_copy(acc_v, out_hbm.at[wid])


_sc_call = functools.partial(
    pl.kernel,
    out_type=jax.ShapeDtypeStruct((NW, ROWS, 128), jnp.float32),
    mesh=plsc.VectorSubcoreMesh(core_axis_name="c", subcore_axis_name="s",
                                num_cores=NC, num_subcores=NS),
    compiler_params=pltpu.CompilerParams(needs_layout_passes=False),
    scratch_types=[
        pltpu.VMEM((ROWS, 128), jnp.float32),
        pltpu.VMEM((N_NODES // 4,), jnp.int32),
        pltpu.VMEM((64,), jnp.float32),
        pltpu.VMEM((64,), jnp.float32),
        pltpu.VMEM((64,), jnp.float32),
        pltpu.VMEM((64,), jnp.float32),
        pltpu.VMEM((CHUNK,), jnp.int32),
        pltpu.VMEM((CHUNK,), jnp.int32),
        pltpu.VMEM((CHUNK,), jnp.float32),
        pltpu.VMEM((CHUNK,), jnp.float32),
    ],
)(_sc_body)


def _combine_body(p_ref, o_ref):
    o_ref[...] = jnp.sum(p_ref[...], axis=0)


_combine = pl.pallas_call(
    _combine_body,
    grid=(ROWS // 112,),
    in_specs=[pl.BlockSpec((NW, 112, 128), lambda i: (0, i, 0))],
    out_specs=pl.BlockSpec((112, 128), lambda i: (i, 0)),
    out_shape=jax.ShapeDtypeStruct((ROWS, 128), jnp.float32),
)


def kernel(edge_index, atom_types, edge_lengths, edge_cutoff, sigma, delta, epsilon):
    a = atom_types.reshape(N_NODES // 4, 4)
    tp = (a[:, 0] | (a[:, 1] << 8) | (a[:, 2] << 16) | (a[:, 3] << 24))
    partials = _sc_call(edge_index[0], edge_index[1], tp, edge_lengths,
                        edge_cutoff.reshape(-1), sigma.reshape(-1),
                        delta.reshape(-1), epsilon.reshape(-1))
    comb = _combine(partials)
    return comb.reshape(-1)[:N_NODES].reshape(-1, 1)
